# baseline (speedup 1.0000x reference)
import functools

import jax
import jax.numpy as jnp
from jax import lax
from jax.experimental import pallas as pl
from jax.experimental.pallas import tpu as pltpu
from jax.experimental.pallas import tpu_sc as plsc

_B = 16384
_NP = 30
_F = 4
_NCOMB = (_NP * (_NP - 1)) // 2
_PLANE = _B * _F
_NW = 32
_SL = _PLANE // _NW
_G = 5
_NB = _NCOMB // _G
_VPC = _SL // 16
_LANES = 16


def _sc_call(xp):
    mesh = plsc.VectorSubcoreMesh(core_axis_name="c", subcore_axis_name="s")

    @functools.partial(
        pl.kernel,
        mesh=mesh,
        compiler_params=pltpu.CompilerParams(needs_layout_passes=False),
        out_type=jax.ShapeDtypeStruct((_NCOMB * _PLANE,), jnp.float32),
        scratch_types=[
            pltpu.VMEM((_NP * _SL,), jnp.float32),
            pltpu.VMEM((2 * _G * _SL,), jnp.float32),
            pltpu.SemaphoreType.DMA,
            pltpu.SemaphoreType.DMA,
        ],
    )
    def k(x_hbm, out_hbm, xs_v, ob_v, sem0, sem1):
        wid = lax.axis_index("s") * 2 + lax.axis_index("c")
        col0 = wid * _SL

        for p in range(_NP):
            pltpu.make_async_copy(
                x_hbm.at[pl.ds(p * _PLANE + col0, _SL)],
                xs_v.at[pl.ds(p * _SL, _SL)],
                sem0,
            ).start()
        for p in range(_NP):
            pltpu.make_async_copy(
                x_hbm.at[pl.ds(p * _PLANE + col0, _SL)],
                xs_v.at[pl.ds(p * _SL, _SL)],
                sem0,
            ).wait()

        def compute_batch(ij, slot):
            i, j = ij
            pairs = []
            anyw = None
            for kk in range(_G):
                pairs.append((i, j))
                j2 = j + 1
                w = j2 >= _NP
                if kk < _G - 1:
                    anyw = w if anyw is None else jnp.logical_or(anyw, w)
                i = jnp.where(w, i + 1, i)
                j = jnp.where(w, i + 1, j2)

            obs = [(slot * _G + kk) * _SL for kk in range(_G)]

            @pl.when(jnp.logical_not(anyw))
            def _():
                ib = pairs[0][0] * _SL
                jb0 = pairs[0][1] * _SL

                @plsc.parallel_loop(0, _VPC, unroll=16)
                def vb(v):
                    o = pl.multiple_of(v * _LANES, _LANES)
                    a = xs_v[pl.ds(ib + o, _LANES)]
                    for kk in range(_G):
                        ob_v[pl.ds(obs[kk] + o, _LANES)] = a + xs_v[
                            pl.ds(jb0 + kk * _SL + o, _LANES)
                        ]

            @pl.when(anyw)
            def _():
                for kk in range(_G):
                    ibase = pairs[kk][0] * _SL
                    jbase = pairs[kk][1] * _SL
                    sbase = obs[kk]

                    @plsc.parallel_loop(0, _VPC, unroll=16)
                    def vbody(v):
                        o = pl.multiple_of(v * _LANES, _LANES)
                        ob_v[pl.ds(sbase + o, _LANES)] = (
                            xs_v[pl.ds(ibase + o, _LANES)]
                            + xs_v[pl.ds(jbase + o, _LANES)]
                        )

            return (i, j)

        def dma_copies(m, slot, sem):
            for kk in range(_G):
                yield pltpu.make_async_copy(
                    ob_v.at[pl.ds((slot * _G + kk) * _SL, _SL)],
                    out_hbm.at[pl.ds((m * _G + kk) * _PLANE + col0, _SL)],
                    sem,
                )

        def dma_start(m, slot, sem):
            for cp in dma_copies(m, slot, sem):
                cp.start()

        def dma_wait(m, slot, sem):
            for cp in dma_copies(m, slot, sem):
                cp.wait()

        ij = (jnp.int32(0), jnp.int32(1))
        ij = compute_batch(ij, 0)
        dma_start(0, 0, sem0)
        ij = compute_batch(ij, 1)
        dma_start(1, 1, sem1)

        def body(t, ij):
            m0 = 2 * t
            dma_wait(m0 - 2, 0, sem0)
            ij = compute_batch(ij, 0)
            dma_start(m0, 0, sem0)
            dma_wait(m0 - 1, 1, sem1)
            ij = compute_batch(ij, 1)
            dma_start(m0 + 1, 1, sem1)
            return ij

        ij = lax.fori_loop(1, _NB // 2, body, ij)
        m_last = _NB - 1
        dma_wait(m_last - 2, 0, sem0)
        ij = compute_batch(ij, 0)
        dma_start(m_last, 0, sem0)
        dma_wait(m_last, 0, sem0)
        dma_wait(m_last - 1, 1, sem1)

    return k(xp)


def kernel(x, combos):
    del combos
    xp = (
        x.reshape(_B // 128, 128, _NP, _F)
        .transpose((2, 0, 3, 1))
        .reshape(_NP * _PLANE)
    )
    r = _sc_call(xp)
    return (
        r.reshape(_NCOMB, _B // 128, _F, 128)
        .transpose((1, 3, 0, 2))
        .reshape(_B, _NCOMB, _F)
    )


# speedup vs baseline: 1.0217x; 1.0217x over previous
import functools

import jax
import jax.numpy as jnp
from jax import lax
from jax.experimental import pallas as pl
from jax.experimental.pallas import tpu as pltpu
from jax.experimental.pallas import tpu_sc as plsc

_B = 16384
_NP = 30
_F = 4
_NCOMB = (_NP * (_NP - 1)) // 2
_PLANE = _B * _F
_NW = 32
_SL = _PLANE // _NW
_G = 5
_NB = _NCOMB // _G
_VPC = _SL // 16
_LANES = 16


def _sc_call(xp):
    mesh = plsc.VectorSubcoreMesh(core_axis_name="c", subcore_axis_name="s")

    @functools.partial(
        pl.kernel,
        mesh=mesh,
        compiler_params=pltpu.CompilerParams(needs_layout_passes=False),
        out_type=jax.ShapeDtypeStruct((_NCOMB * _PLANE,), jnp.float32),
        scratch_types=[
            pltpu.VMEM((_NP * _SL,), jnp.float32),
            pltpu.VMEM((2 * _G * _SL,), jnp.float32),
            pltpu.SemaphoreType.DMA,
            pltpu.SemaphoreType.DMA,
        ],
    )
    def k(x_hbm, out_hbm, xs_v, ob_v, sem0, sem1):
        wid = lax.axis_index("s") * 2 + lax.axis_index("c")
        col0 = wid * _SL

        for p in range(_NP):
            pltpu.make_async_copy(
                x_hbm.at[pl.ds(p * _PLANE + col0, _SL)],
                xs_v.at[pl.ds(p * _SL, _SL)],
                sem0,
            ).start()
        for p in range(_NP):
            pltpu.make_async_copy(
                x_hbm.at[pl.ds(p * _PLANE + col0, _SL)],
                xs_v.at[pl.ds(p * _SL, _SL)],
                sem0,
            ).wait()

        def compute_batch(ij, slot):
            i, j = ij
            pairs = []
            anyw = None
            for kk in range(_G):
                pairs.append((i, j))
                j2 = j + 1
                w = j2 >= _NP
                if kk < _G - 1:
                    anyw = w if anyw is None else jnp.logical_or(anyw, w)
                i = jnp.where(w, i + 1, i)
                j = jnp.where(w, i + 1, j2)

            obs = [(slot * _G + kk) * _SL for kk in range(_G)]

            @pl.when(jnp.logical_not(anyw))
            def _():
                ib = pairs[0][0] * _SL
                jb0 = pairs[0][1] * _SL

                @plsc.parallel_loop(0, _VPC, unroll=8)
                def vb(v):
                    o = pl.multiple_of(v * _LANES, _LANES)
                    a = xs_v[pl.ds(ib + o, _LANES)]
                    for kk in range(_G):
                        ob_v[pl.ds(obs[kk] + o, _LANES)] = a + xs_v[
                            pl.ds(jb0 + kk * _SL + o, _LANES)
                        ]

            @pl.when(anyw)
            def _():
                for kk in range(_G):
                    ibase = pairs[kk][0] * _SL
                    jbase = pairs[kk][1] * _SL
                    sbase = obs[kk]

                    @plsc.parallel_loop(0, _VPC, unroll=8)
                    def vbody(v):
                        o = pl.multiple_of(v * _LANES, _LANES)
                        ob_v[pl.ds(sbase + o, _LANES)] = (
                            xs_v[pl.ds(ibase + o, _LANES)]
                            + xs_v[pl.ds(jbase + o, _LANES)]
                        )

            return (i, j)

        def dma_copies(m, slot, sem):
            for kk in range(_G):
                yield pltpu.make_async_copy(
                    ob_v.at[pl.ds((slot * _G + kk) * _SL, _SL)],
                    out_hbm.at[pl.ds((m * _G + kk) * _PLANE + col0, _SL)],
                    sem,
                )

        def dma_start(m, slot, sem):
            for cp in dma_copies(m, slot, sem):
                cp.start()

        def dma_wait(m, slot, sem):
            for cp in dma_copies(m, slot, sem):
                cp.wait()

        ij = (jnp.int32(0), jnp.int32(1))
        ij = compute_batch(ij, 0)
        dma_start(0, 0, sem0)
        ij = compute_batch(ij, 1)
        dma_start(1, 1, sem1)

        def body(t, ij):
            m0 = 2 * t
            dma_wait(m0 - 2, 0, sem0)
            ij = compute_batch(ij, 0)
            dma_start(m0, 0, sem0)
            dma_wait(m0 - 1, 1, sem1)
            ij = compute_batch(ij, 1)
            dma_start(m0 + 1, 1, sem1)
            return ij

        ij = lax.fori_loop(1, _NB // 2, body, ij)
        m_last = _NB - 1
        dma_wait(m_last - 2, 0, sem0)
        ij = compute_batch(ij, 0)
        dma_start(m_last, 0, sem0)
        dma_wait(m_last, 0, sem0)
        dma_wait(m_last - 1, 1, sem1)

    return k(xp)


def kernel(x, combos):
    del combos
    xp = (
        x.reshape(_B // 128, 128, _NP, _F)
        .transpose((2, 0, 3, 1))
        .reshape(_NP * _PLANE)
    )
    r = _sc_call(xp)
    return (
        r.reshape(_NCOMB, _B // 128, _F, 128)
        .transpose((1, 3, 0, 2))
        .reshape(_B, _NCOMB, _F)
    )
